# baseline (speedup 1.0000x reference)
import functools

import jax
import jax.numpy as jnp
from jax.experimental import pallas as pl
from jax.experimental.pallas import tpu as pltpu


def _stem_kernel(a_ref, b_ref, bias_ref, o_ref):
    r = jnp.dot(a_ref[...], b_ref[...], preferred_element_type=jnp.float32)
    r = r + bias_ref[...]
    r = r * jax.nn.sigmoid(r)
    o_ref[...] = r.astype(o_ref.dtype)


def _stem_matmul(patches, w_stem, b_stem):
    M, K = patches.shape
    N = w_stem.shape[1]
    b_stem = b_stem.reshape(1, N)
    tm = min(512, M)
    grid = (M // tm,)
    return pl.pallas_call(
        _stem_kernel,
        out_shape=jax.ShapeDtypeStruct((M, N), jnp.bfloat16),
        grid=grid,
        in_specs=[
            pl.BlockSpec((tm, K), lambda m: (m, 0)),
            pl.BlockSpec((K, N), lambda m: (0, 0)),
            pl.BlockSpec((1, N), lambda m: (0, 0)),
        ],
        out_specs=pl.BlockSpec((tm, N), lambda m: (m, 0)),
        compiler_params=pltpu.CompilerParams(
            dimension_semantics=("parallel",),
            vmem_limit_bytes=56 * 1024 * 1024),
        cost_estimate=pl.CostEstimate(
            flops=2 * M * K * N, transcendentals=M * N,
            bytes_accessed=M * K * 2 + K * N * 2 + M * N * 2),
    )(patches, w_stem, b_stem)


def _deconv_kernel(aw_ref, b_ref, bias_ref, o_ref, *, H, W):
    C3 = aw_ref.shape[3]
    acc = jnp.dot(aw_ref[0, 0:H].reshape(H * W, C3), b_ref[0:C3],
                  preferred_element_type=jnp.float32)
    acc += jnp.dot(aw_ref[0, 1:H + 1].reshape(H * W, C3), b_ref[C3:2 * C3],
                   preferred_element_type=jnp.float32)
    acc += jnp.dot(aw_ref[0, 2:H + 2].reshape(H * W, C3), b_ref[2 * C3:3 * C3],
                   preferred_element_type=jnp.float32)
    r = jnp.maximum(acc + bias_ref[...], 0.0)
    o_ref[0] = r.astype(o_ref.dtype)


def _deconv_layer(aw, B9, bias4, H, W, n_split):
    N = aw.shape[0]
    C3 = aw.shape[3]
    NC = B9.shape[1]
    tn = NC // n_split
    bias_row = bias4.reshape(1, NC)
    grid = (n_split, N)
    kern = functools.partial(_deconv_kernel, H=H, W=W)
    flops = 2 * N * H * W * 3 * C3 * NC
    bytes_accessed = (N * (H + 2) * W * C3 * 2 * n_split + 3 * C3 * NC * 2
                      + N * H * W * NC * 2)
    return pl.pallas_call(
        kern,
        out_shape=jax.ShapeDtypeStruct((N, H * W, NC), jnp.bfloat16),
        grid=grid,
        in_specs=[
            pl.BlockSpec((1, H + 2, W, C3), lambda n, i: (i, 0, 0, 0)),
            pl.BlockSpec((3 * C3, tn), lambda n, i: (0, n)),
            pl.BlockSpec((1, tn), lambda n, i: (0, n)),
        ],
        out_specs=pl.BlockSpec((1, H * W, tn), lambda n, i: (i, 0, n)),
        compiler_params=pltpu.CompilerParams(
            dimension_semantics=("parallel", "arbitrary"),
            vmem_limit_bytes=56 * 1024 * 1024),
        cost_estimate=pl.CostEstimate(
            flops=flops, transcendentals=0,
            bytes_accessed=int(bytes_accessed)),
    )(aw, B9, bias_row)


def _deconv_head_kernel(aw_ref, b_ref, bias_ref, w14_ref, o_ref, *, H, W):
    C3 = aw_ref.shape[3]
    acc = jnp.dot(aw_ref[0, 0:H].reshape(H * W, C3), b_ref[0:C3],
                  preferred_element_type=jnp.float32)
    acc += jnp.dot(aw_ref[0, 1:H + 1].reshape(H * W, C3), b_ref[C3:2 * C3],
                   preferred_element_type=jnp.float32)
    acc += jnp.dot(aw_ref[0, 2:H + 2].reshape(H * W, C3), b_ref[2 * C3:3 * C3],
                   preferred_element_type=jnp.float32)
    r = jnp.maximum(acc + bias_ref[...], 0.0).astype(jnp.bfloat16)
    g = jnp.dot(r, w14_ref[...], preferred_element_type=jnp.float32)
    o_ref[0] = g.astype(o_ref.dtype)


def _deconv_head_layer(aw, B9, bias4, w14, H, W):
    N = aw.shape[0]
    C3 = aw.shape[3]
    NC = B9.shape[1]
    GC = w14.shape[1]
    bias_row = bias4.reshape(1, NC)
    kern = functools.partial(_deconv_head_kernel, H=H, W=W)
    flops = 2 * N * H * W * (3 * C3 * NC + NC * GC)
    bytes_accessed = (N * (H + 2) * W * C3 * 2 + 3 * C3 * NC * 2
                      + N * H * W * GC * 2)
    return pl.pallas_call(
        kern,
        out_shape=jax.ShapeDtypeStruct((N, H * W, GC), jnp.bfloat16),
        grid=(N,),
        in_specs=[
            pl.BlockSpec((1, H + 2, W, C3), lambda i: (i, 0, 0, 0)),
            pl.BlockSpec((3 * C3, NC), lambda i: (0, 0)),
            pl.BlockSpec((1, NC), lambda i: (0, 0)),
            pl.BlockSpec((NC, GC), lambda i: (0, 0)),
        ],
        out_specs=pl.BlockSpec((1, H * W, GC), lambda i: (i, 0, 0)),
        compiler_params=pltpu.CompilerParams(
            dimension_semantics=("parallel",),
            vmem_limit_bytes=56 * 1024 * 1024),
        cost_estimate=pl.CostEstimate(
            flops=flops, transcendentals=0,
            bytes_accessed=int(bytes_accessed)),
    )(aw, B9, bias_row, w14)


def _resize_kernel(g_ref, whp0_ref, whp1_ref, wwp_ref, bias_ref, o_ref):
    c = pl.program_id(1)
    g = g_ref[0, 0]
    c0 = jnp.concatenate([g[0], g[1]], axis=-1)
    c1 = jnp.concatenate([g[2], g[3]], axis=-1)
    t = (jnp.dot(whp0_ref[...], c0, preferred_element_type=jnp.float32)
         + jnp.dot(whp1_ref[...], c1, preferred_element_type=jnp.float32))
    y = jnp.dot(t.astype(jnp.bfloat16), wwp_ref[...],
                preferred_element_type=jnp.float32)
    o_ref[0, 0] = y + bias_ref[c]


def _head_resize(ypp, wh, wwT, bias):
    N, C, _, h2, w2 = ypp.shape
    OH = wh.shape[0]
    OW = wwT.shape[1]
    whp0 = wh[:, 0::2]
    whp1 = wh[:, 1::2]
    wwp = jnp.concatenate([wwT[0::2], wwT[1::2]], axis=0)
    flops = 2 * N * C * (2 * OH * h2 * w2 * 2 + OH * 2 * w2 * OW)
    bytes_accessed = (N * C * 4 * h2 * w2 * 2 + 2 * OH * h2 * 2
                      + 2 * w2 * OW * 2 + N * C * OH * OW * 4)
    return pl.pallas_call(
        _resize_kernel,
        out_shape=jax.ShapeDtypeStruct((N, C, OH, OW), jnp.float32),
        grid=(N, C),
        in_specs=[
            pl.BlockSpec((1, 1, 4, h2, w2), lambda n, c: (n, c, 0, 0, 0)),
            pl.BlockSpec((OH, h2), lambda n, c: (0, 0)),
            pl.BlockSpec((OH, h2), lambda n, c: (0, 0)),
            pl.BlockSpec((2 * w2, OW), lambda n, c: (0, 0)),
            pl.BlockSpec(memory_space=pltpu.MemorySpace.SMEM),
        ],
        out_specs=pl.BlockSpec((1, 1, OH, OW), lambda n, c: (n, c, 0, 0)),
        compiler_params=pltpu.CompilerParams(
            dimension_semantics=("parallel", "parallel"),
            vmem_limit_bytes=56 * 1024 * 1024),
        cost_estimate=pl.CostEstimate(
            flops=flops, transcendentals=0,
            bytes_accessed=int(bytes_accessed)),
    )(ypp, whp0, whp1, wwp, bias.astype(jnp.float32))


def _width_cat(x_nhwc):
    xp = jnp.pad(x_nhwc, ((0, 0), (1, 1), (1, 1), (0, 0)))
    W = x_nhwc.shape[2]
    return jnp.concatenate([xp[:, :, b:b + W, :] for b in range(3)], axis=-1)


def _depth_to_space(y, H, W, C):
    N = y.shape[0]
    return (y.reshape(N, H, W, 2, 2, C)
             .transpose(0, 1, 3, 2, 4, 5)
             .reshape(N, 2 * H, 2 * W, C))


def kernel(x, w_stem, b_stem, w1x1, b1x1, Wh, WwT,
           B_t1, bias4_t1, B_t2, bias4_t2, B_t3, bias4_t3):
    N, C, H, W = x.shape
    P = 32
    hp, wp = H // P, W // P

    patches = (x.astype(jnp.bfloat16)
               .reshape(N, C, hp, P, wp, P)
               .transpose(0, 2, 4, 1, 3, 5)
               .reshape(N * hp * wp, C * P * P))
    f0 = _stem_matmul(patches, w_stem, b_stem).reshape(N, hp, wp, -1)

    aw1 = _width_cat(f0)
    y1 = _deconv_layer(aw1, B_t1, bias4_t1, hp, wp, n_split=2)
    f1 = _depth_to_space(y1, hp, wp, B_t1.shape[1] // 4)

    aw2 = _width_cat(f1)
    y2 = _deconv_layer(aw2, B_t2, bias4_t2, 2 * hp, 2 * wp, n_split=1)
    f2 = _depth_to_space(y2, 2 * hp, 2 * wp, B_t2.shape[1] // 4)

    aw3 = _width_cat(f2)
    nclass = w1x1.shape[1]
    cout3 = B_t3.shape[1] // 4
    w14 = jnp.zeros((4 * cout3, 4 * nclass), jnp.bfloat16)
    for ph in range(4):
        w14 = w14.at[ph * cout3:(ph + 1) * cout3, ph::4].set(w1x1)
    y3 = _deconv_head_layer(aw3, B_t3, bias4_t3, w14, 4 * hp, 4 * wp)

    ypp = (y3.reshape(N, 4 * hp, 4 * wp, nclass, 4)
             .transpose(0, 3, 4, 1, 2))
    return _head_resize(ypp, Wh, WwT, b1x1)


# speedup vs baseline: 1.5185x; 1.5185x over previous
import functools

import jax
import jax.numpy as jnp
from jax.experimental import pallas as pl
from jax.experimental.pallas import tpu as pltpu


def _stem_kernel(a_ref, b_ref, bias_ref, o_ref):
    r = jnp.dot(a_ref[...], b_ref[...], preferred_element_type=jnp.float32)
    r = r + bias_ref[...]
    r = r * jax.nn.sigmoid(r)
    o_ref[...] = r.astype(o_ref.dtype)


def _stem_matmul(patches, w_stem, b_stem):
    M, K = patches.shape
    N = w_stem.shape[1]
    b_stem = b_stem.reshape(1, N)
    tm = min(512, M)
    grid = (M // tm,)
    return pl.pallas_call(
        _stem_kernel,
        out_shape=jax.ShapeDtypeStruct((M, N), jnp.bfloat16),
        grid=grid,
        in_specs=[
            pl.BlockSpec((tm, K), lambda m: (m, 0)),
            pl.BlockSpec((K, N), lambda m: (0, 0)),
            pl.BlockSpec((1, N), lambda m: (0, 0)),
        ],
        out_specs=pl.BlockSpec((tm, N), lambda m: (m, 0)),
        compiler_params=pltpu.CompilerParams(
            dimension_semantics=("parallel",),
            vmem_limit_bytes=56 * 1024 * 1024),
        cost_estimate=pl.CostEstimate(
            flops=2 * M * K * N, transcendentals=M * N,
            bytes_accessed=M * K * 2 + K * N * 2 + M * N * 2),
    )(patches, w_stem, b_stem)


def _deconv_kernel(aw_ref, b_ref, bias_ref, o_ref, *, H, W):
    C3 = aw_ref.shape[3]
    acc = jnp.dot(aw_ref[0, 0:H].reshape(H * W, C3), b_ref[0:C3],
                  preferred_element_type=jnp.float32)
    acc += jnp.dot(aw_ref[0, 1:H + 1].reshape(H * W, C3), b_ref[C3:2 * C3],
                   preferred_element_type=jnp.float32)
    acc += jnp.dot(aw_ref[0, 2:H + 2].reshape(H * W, C3), b_ref[2 * C3:3 * C3],
                   preferred_element_type=jnp.float32)
    r = jnp.maximum(acc + bias_ref[...], 0.0)
    o_ref[0] = r.astype(o_ref.dtype)


def _deconv_layer(aw, B9, bias4, H, W, n_split):
    N = aw.shape[0]
    C3 = aw.shape[3]
    NC = B9.shape[1]
    tn = NC // n_split
    bias_row = bias4.reshape(1, NC)
    grid = (n_split, N)
    kern = functools.partial(_deconv_kernel, H=H, W=W)
    flops = 2 * N * H * W * 3 * C3 * NC
    bytes_accessed = (N * (H + 2) * W * C3 * 2 * n_split + 3 * C3 * NC * 2
                      + N * H * W * NC * 2)
    return pl.pallas_call(
        kern,
        out_shape=jax.ShapeDtypeStruct((N, H * W, NC), jnp.bfloat16),
        grid=grid,
        in_specs=[
            pl.BlockSpec((1, H + 2, W, C3), lambda n, i: (i, 0, 0, 0)),
            pl.BlockSpec((3 * C3, tn), lambda n, i: (0, n)),
            pl.BlockSpec((1, tn), lambda n, i: (0, n)),
        ],
        out_specs=pl.BlockSpec((1, H * W, tn), lambda n, i: (i, 0, n)),
        compiler_params=pltpu.CompilerParams(
            dimension_semantics=("parallel", "arbitrary"),
            vmem_limit_bytes=56 * 1024 * 1024),
        cost_estimate=pl.CostEstimate(
            flops=flops, transcendentals=0,
            bytes_accessed=int(bytes_accessed)),
    )(aw, B9, bias_row)


def _deconv_head_kernel(aw_ref, b_ref, bias_ref, w14_ref, o_ref, *, H, W):
    C3 = aw_ref.shape[3]
    acc = jnp.dot(aw_ref[0, 0:H].reshape(H * W, C3), b_ref[0:C3],
                  preferred_element_type=jnp.float32)
    acc += jnp.dot(aw_ref[0, 1:H + 1].reshape(H * W, C3), b_ref[C3:2 * C3],
                   preferred_element_type=jnp.float32)
    acc += jnp.dot(aw_ref[0, 2:H + 2].reshape(H * W, C3), b_ref[2 * C3:3 * C3],
                   preferred_element_type=jnp.float32)
    r = jnp.maximum(acc + bias_ref[...], 0.0).astype(jnp.bfloat16)
    g = jnp.dot(r, w14_ref[...], preferred_element_type=jnp.float32)
    o_ref[0] = g.astype(o_ref.dtype)


def _deconv_head_layer(aw, B9, bias4, w14, H, W):
    N = aw.shape[0]
    C3 = aw.shape[3]
    NC = B9.shape[1]
    GC = w14.shape[1]
    bias_row = bias4.reshape(1, NC)
    kern = functools.partial(_deconv_head_kernel, H=H, W=W)
    flops = 2 * N * H * W * (3 * C3 * NC + NC * GC)
    bytes_accessed = (N * (H + 2) * W * C3 * 2 + 3 * C3 * NC * 2
                      + N * H * W * GC * 2)
    return pl.pallas_call(
        kern,
        out_shape=jax.ShapeDtypeStruct((N, H * W, GC), jnp.bfloat16),
        grid=(N,),
        in_specs=[
            pl.BlockSpec((1, H + 2, W, C3), lambda i: (i, 0, 0, 0)),
            pl.BlockSpec((3 * C3, NC), lambda i: (0, 0)),
            pl.BlockSpec((1, NC), lambda i: (0, 0)),
            pl.BlockSpec((NC, GC), lambda i: (0, 0)),
        ],
        out_specs=pl.BlockSpec((1, H * W, GC), lambda i: (i, 0, 0)),
        compiler_params=pltpu.CompilerParams(
            dimension_semantics=("parallel",),
            vmem_limit_bytes=56 * 1024 * 1024),
        cost_estimate=pl.CostEstimate(
            flops=flops, transcendentals=0,
            bytes_accessed=int(bytes_accessed)),
    )(aw, B9, bias_row, w14)


def _resize_kernel(g_ref, whp0_ref, whp1_ref, wwp_ref, bias_ref, o_ref):
    c = pl.program_id(1)
    g = g_ref[0, 0]
    c0 = jnp.concatenate([g[0], g[1]], axis=-1)
    c1 = jnp.concatenate([g[2], g[3]], axis=-1)
    t = (jnp.dot(whp0_ref[...], c0, preferred_element_type=jnp.float32)
         + jnp.dot(whp1_ref[...], c1, preferred_element_type=jnp.float32))
    y = jnp.dot(t.astype(jnp.bfloat16), wwp_ref[...],
                preferred_element_type=jnp.float32)
    o_ref[0, 0] = y + bias_ref[c]


def _head_resize(ypp, wh, wwT, bias):
    N, C, _, h2, w2 = ypp.shape
    OH = wh.shape[0]
    OW = wwT.shape[1]
    whT = wh.reshape(OH, h2, 2).transpose(2, 0, 1)
    whp0 = whT[0]
    whp1 = whT[1]
    wwp = (wwT.reshape(w2, 2, OW).transpose(1, 0, 2)
           .reshape(2 * w2, OW))
    flops = 2 * N * C * (2 * OH * h2 * w2 * 2 + OH * 2 * w2 * OW)
    bytes_accessed = (N * C * 4 * h2 * w2 * 2 + 2 * OH * h2 * 2
                      + 2 * w2 * OW * 2 + N * C * OH * OW * 4)
    return pl.pallas_call(
        _resize_kernel,
        out_shape=jax.ShapeDtypeStruct((N, C, OH, OW), jnp.float32),
        grid=(N, C),
        in_specs=[
            pl.BlockSpec((1, 1, 4, h2, w2), lambda n, c: (n, c, 0, 0, 0)),
            pl.BlockSpec((OH, h2), lambda n, c: (0, 0)),
            pl.BlockSpec((OH, h2), lambda n, c: (0, 0)),
            pl.BlockSpec((2 * w2, OW), lambda n, c: (0, 0)),
            pl.BlockSpec(memory_space=pltpu.MemorySpace.SMEM),
        ],
        out_specs=pl.BlockSpec((1, 1, OH, OW), lambda n, c: (n, c, 0, 0)),
        compiler_params=pltpu.CompilerParams(
            dimension_semantics=("parallel", "parallel"),
            vmem_limit_bytes=56 * 1024 * 1024),
        cost_estimate=pl.CostEstimate(
            flops=flops, transcendentals=0,
            bytes_accessed=int(bytes_accessed)),
    )(ypp, whp0, whp1, wwp, bias.astype(jnp.float32))


def _width_cat(x_nhwc):
    xp = jnp.pad(x_nhwc, ((0, 0), (1, 1), (1, 1), (0, 0)))
    W = x_nhwc.shape[2]
    return jnp.concatenate([xp[:, :, b:b + W, :] for b in range(3)], axis=-1)


def _depth_to_space(y, H, W, C):
    N = y.shape[0]
    return (y.reshape(N, H, W, 2, 2, C)
             .transpose(0, 1, 3, 2, 4, 5)
             .reshape(N, 2 * H, 2 * W, C))


def kernel(x, w_stem, b_stem, w1x1, b1x1, Wh, WwT,
           B_t1, bias4_t1, B_t2, bias4_t2, B_t3, bias4_t3):
    N, C, H, W = x.shape
    P = 32
    hp, wp = H // P, W // P

    patches = (x.astype(jnp.bfloat16)
               .reshape(N, C, hp, P, wp, P)
               .transpose(0, 2, 4, 1, 3, 5)
               .reshape(N * hp * wp, C * P * P))
    f0 = _stem_matmul(patches, w_stem, b_stem).reshape(N, hp, wp, -1)

    aw1 = _width_cat(f0)
    y1 = _deconv_layer(aw1, B_t1, bias4_t1, hp, wp, n_split=2)
    f1 = _depth_to_space(y1, hp, wp, B_t1.shape[1] // 4)

    aw2 = _width_cat(f1)
    y2 = _deconv_layer(aw2, B_t2, bias4_t2, 2 * hp, 2 * wp, n_split=1)
    f2 = _depth_to_space(y2, 2 * hp, 2 * wp, B_t2.shape[1] // 4)

    aw3 = _width_cat(f2)
    nclass = w1x1.shape[1]
    cout3 = B_t3.shape[1] // 4
    eye4 = jnp.eye(4, dtype=jnp.bfloat16)
    w14 = (eye4[:, None, None, :] * w1x1[None, :, :, None]
           ).reshape(4 * cout3, 4 * nclass)
    y3 = _deconv_head_layer(aw3, B_t3, bias4_t3, w14, 4 * hp, 4 * wp)

    ypp = (y3.reshape(N, 4 * hp, 4 * wp, nclass, 4)
             .transpose(0, 3, 4, 1, 2))
    return _head_resize(ypp, Wh, WwT, b1x1)


# speedup vs baseline: 1.5199x; 1.0009x over previous
import functools

import jax
import jax.numpy as jnp
from jax.experimental import pallas as pl
from jax.experimental.pallas import tpu as pltpu


def _stem_kernel(a_ref, b_ref, bias_ref, o_ref):
    r = jnp.dot(a_ref[...], b_ref[...], preferred_element_type=jnp.float32)
    r = r + bias_ref[...]
    r = r * jax.nn.sigmoid(r)
    o_ref[...] = r.astype(o_ref.dtype)


def _stem_matmul(patches, w_stem, b_stem):
    M, K = patches.shape
    N = w_stem.shape[1]
    b_stem = b_stem.reshape(1, N)
    tm = min(512, M)
    grid = (M // tm,)
    return pl.pallas_call(
        _stem_kernel,
        out_shape=jax.ShapeDtypeStruct((M, N), jnp.bfloat16),
        grid=grid,
        in_specs=[
            pl.BlockSpec((tm, K), lambda m: (m, 0)),
            pl.BlockSpec((K, N), lambda m: (0, 0)),
            pl.BlockSpec((1, N), lambda m: (0, 0)),
        ],
        out_specs=pl.BlockSpec((tm, N), lambda m: (m, 0)),
        compiler_params=pltpu.CompilerParams(
            dimension_semantics=("parallel",),
            vmem_limit_bytes=56 * 1024 * 1024),
        cost_estimate=pl.CostEstimate(
            flops=2 * M * K * N, transcendentals=M * N,
            bytes_accessed=M * K * 2 + K * N * 2 + M * N * 2),
    )(patches, w_stem, b_stem)


def _deconv_kernel(aw_ref, b_ref, bias_ref, o_ref, *, H, W):
    C3 = aw_ref.shape[3]
    acc = jnp.dot(aw_ref[0, 0:H].reshape(H * W, C3), b_ref[0:C3],
                  preferred_element_type=jnp.float32)
    acc += jnp.dot(aw_ref[0, 1:H + 1].reshape(H * W, C3), b_ref[C3:2 * C3],
                   preferred_element_type=jnp.float32)
    acc += jnp.dot(aw_ref[0, 2:H + 2].reshape(H * W, C3), b_ref[2 * C3:3 * C3],
                   preferred_element_type=jnp.float32)
    r = jnp.maximum(acc + bias_ref[...], 0.0)
    o_ref[0] = r.astype(o_ref.dtype)


def _deconv_layer(aw, B9, bias4, H, W, n_split):
    N = aw.shape[0]
    C3 = aw.shape[3]
    NC = B9.shape[1]
    tn = NC // n_split
    bias_row = bias4.reshape(1, NC)
    grid = (n_split, N)
    kern = functools.partial(_deconv_kernel, H=H, W=W)
    flops = 2 * N * H * W * 3 * C3 * NC
    bytes_accessed = (N * (H + 2) * W * C3 * 2 * n_split + 3 * C3 * NC * 2
                      + N * H * W * NC * 2)
    return pl.pallas_call(
        kern,
        out_shape=jax.ShapeDtypeStruct((N, H * W, NC), jnp.bfloat16),
        grid=grid,
        in_specs=[
            pl.BlockSpec((1, H + 2, W, C3), lambda n, i: (i, 0, 0, 0)),
            pl.BlockSpec((3 * C3, tn), lambda n, i: (0, n)),
            pl.BlockSpec((1, tn), lambda n, i: (0, n)),
        ],
        out_specs=pl.BlockSpec((1, H * W, tn), lambda n, i: (i, 0, n)),
        compiler_params=pltpu.CompilerParams(
            dimension_semantics=("parallel", "arbitrary"),
            vmem_limit_bytes=56 * 1024 * 1024),
        cost_estimate=pl.CostEstimate(
            flops=flops, transcendentals=0,
            bytes_accessed=int(bytes_accessed)),
    )(aw, B9, bias_row)


def _deconv_head_kernel(aw_ref, b_ref, bias_ref, w14_ref, o_ref, *, H, W):
    C3 = aw_ref.shape[3]
    acc = jnp.dot(aw_ref[0, 0:H].reshape(H * W, C3), b_ref[0:C3],
                  preferred_element_type=jnp.float32)
    acc += jnp.dot(aw_ref[0, 1:H + 1].reshape(H * W, C3), b_ref[C3:2 * C3],
                   preferred_element_type=jnp.float32)
    acc += jnp.dot(aw_ref[0, 2:H + 2].reshape(H * W, C3), b_ref[2 * C3:3 * C3],
                   preferred_element_type=jnp.float32)
    r = jnp.maximum(acc + bias_ref[...], 0.0).astype(jnp.bfloat16)
    g = jnp.dot(r, w14_ref[...], preferred_element_type=jnp.float32)
    o_ref[0] = g.astype(o_ref.dtype)


def _deconv_head_layer(aw, B9, bias4, w14, H, W):
    N = aw.shape[0]
    C3 = aw.shape[3]
    NC = B9.shape[1]
    GC = w14.shape[1]
    bias_row = bias4.reshape(1, NC)
    kern = functools.partial(_deconv_head_kernel, H=H, W=W)
    flops = 2 * N * H * W * (3 * C3 * NC + NC * GC)
    bytes_accessed = (N * (H + 2) * W * C3 * 2 + 3 * C3 * NC * 2
                      + N * H * W * GC * 2)
    return pl.pallas_call(
        kern,
        out_shape=jax.ShapeDtypeStruct((N, H * W, GC), jnp.bfloat16),
        grid=(N,),
        in_specs=[
            pl.BlockSpec((1, H + 2, W, C3), lambda i: (i, 0, 0, 0)),
            pl.BlockSpec((3 * C3, NC), lambda i: (0, 0)),
            pl.BlockSpec((1, NC), lambda i: (0, 0)),
            pl.BlockSpec((NC, GC), lambda i: (0, 0)),
        ],
        out_specs=pl.BlockSpec((1, H * W, GC), lambda i: (i, 0, 0)),
        compiler_params=pltpu.CompilerParams(
            dimension_semantics=("parallel",),
            vmem_limit_bytes=56 * 1024 * 1024),
        cost_estimate=pl.CostEstimate(
            flops=flops, transcendentals=0,
            bytes_accessed=int(bytes_accessed)),
    )(aw, B9, bias_row, w14)


def _resize_kernel(g_ref, whp0_ref, whp1_ref, wwp_ref, bias_ref, o_ref):
    c = pl.program_id(1)
    g = g_ref[0, 0]
    c0 = jnp.concatenate([g[0], g[1]], axis=-1)
    c1 = jnp.concatenate([g[2], g[3]], axis=-1)
    t = (jnp.dot(whp0_ref[...], c0, preferred_element_type=jnp.float32)
         + jnp.dot(whp1_ref[...], c1, preferred_element_type=jnp.float32))
    y = jnp.dot(t.astype(jnp.bfloat16), wwp_ref[...],
                preferred_element_type=jnp.float32)
    o_ref[0, 0] = y + bias_ref[c]


def _head_resize(ypp, wh, wwT, bias):
    N, C, _, h2, w2 = ypp.shape
    OH = wh.shape[0]
    OW = wwT.shape[1]
    whT = wh.reshape(OH, h2, 2).transpose(2, 0, 1)
    whp0 = whT[0]
    whp1 = whT[1]
    wwp = (wwT.reshape(w2, 2, OW).transpose(1, 0, 2)
           .reshape(2 * w2, OW))
    flops = 2 * N * C * (2 * OH * h2 * w2 * 2 + OH * 2 * w2 * OW)
    bytes_accessed = (N * C * 4 * h2 * w2 * 2 + 2 * OH * h2 * 2
                      + 2 * w2 * OW * 2 + N * C * OH * OW * 4)
    return pl.pallas_call(
        _resize_kernel,
        out_shape=jax.ShapeDtypeStruct((N, C, OH, OW), jnp.float32),
        grid=(N, C),
        in_specs=[
            pl.BlockSpec((1, 1, 4, h2, w2), lambda n, c: (n, c, 0, 0, 0)),
            pl.BlockSpec((OH, h2), lambda n, c: (0, 0)),
            pl.BlockSpec((OH, h2), lambda n, c: (0, 0)),
            pl.BlockSpec((2 * w2, OW), lambda n, c: (0, 0)),
            pl.BlockSpec(memory_space=pltpu.MemorySpace.SMEM),
        ],
        out_specs=pl.BlockSpec((1, 1, OH, OW), lambda n, c: (n, c, 0, 0)),
        compiler_params=pltpu.CompilerParams(
            dimension_semantics=("parallel", "parallel"),
            vmem_limit_bytes=56 * 1024 * 1024),
        cost_estimate=pl.CostEstimate(
            flops=flops, transcendentals=0,
            bytes_accessed=int(bytes_accessed)),
    )(ypp, whp0, whp1, wwp, bias.astype(jnp.float32))


def _width_cat(x_nhwc):
    xp = jnp.pad(x_nhwc, ((0, 0), (1, 1), (1, 1), (0, 0)))
    W = x_nhwc.shape[2]
    return jnp.concatenate([xp[:, :, b:b + W, :] for b in range(3)], axis=-1)


def _depth_to_space(y, H, W, C):
    N = y.shape[0]
    return (y.reshape(N, H, W, 2, 2, C)
             .transpose(0, 1, 3, 2, 4, 5)
             .reshape(N, 2 * H, 2 * W, C))


def kernel(x, w_stem, b_stem, w1x1, b1x1, Wh, WwT,
           B_t1, bias4_t1, B_t2, bias4_t2, B_t3, bias4_t3):
    N, C, H, W = x.shape
    P = 32
    hp, wp = H // P, W // P

    xb = x.astype(jnp.bfloat16).reshape(N, C, hp, P, wp, P)
    x1 = jax.lax.optimization_barrier(xb.transpose(0, 2, 1, 3, 4, 5))
    patches = (x1.transpose(0, 1, 4, 2, 3, 5)
               .reshape(N * hp * wp, C * P * P))
    f0 = _stem_matmul(patches, w_stem, b_stem).reshape(N, hp, wp, -1)

    aw1 = _width_cat(f0)
    y1 = _deconv_layer(aw1, B_t1, bias4_t1, hp, wp, n_split=2)
    f1 = _depth_to_space(y1, hp, wp, B_t1.shape[1] // 4)

    aw2 = _width_cat(f1)
    y2 = _deconv_layer(aw2, B_t2, bias4_t2, 2 * hp, 2 * wp, n_split=1)
    f2 = _depth_to_space(y2, 2 * hp, 2 * wp, B_t2.shape[1] // 4)

    aw3 = _width_cat(f2)
    nclass = w1x1.shape[1]
    cout3 = B_t3.shape[1] // 4
    eye4 = jnp.eye(4, dtype=jnp.bfloat16)
    w14 = (eye4[:, None, None, :] * w1x1[None, :, :, None]
           ).reshape(4 * cout3, 4 * nclass)
    y3 = _deconv_head_layer(aw3, B_t3, bias4_t3, w14, 4 * hp, 4 * wp)

    ypp = (y3.reshape(N, 4 * hp, 4 * wp, nclass, 4)
             .transpose(0, 3, 4, 1, 2))
    return _head_resize(ypp, Wh, WwT, b1x1)


# speedup vs baseline: 1.6445x; 1.0820x over previous
import functools

import jax
import jax.numpy as jnp
from jax.experimental import pallas as pl
from jax.experimental.pallas import tpu as pltpu


def _stem_kernel(a_ref, b_ref, bias_ref, o_ref):
    r = jnp.dot(a_ref[...], b_ref[...], preferred_element_type=jnp.float32)
    r = r + bias_ref[...]
    r = r * jax.nn.sigmoid(r)
    o_ref[...] = r.astype(o_ref.dtype)


def _stem_matmul(patches, w_stem, b_stem):
    M, K = patches.shape
    N = w_stem.shape[1]
    b_stem = b_stem.reshape(1, N)
    tm = min(512, M)
    grid = (M // tm,)
    return pl.pallas_call(
        _stem_kernel,
        out_shape=jax.ShapeDtypeStruct((M, N), jnp.bfloat16),
        grid=grid,
        in_specs=[
            pl.BlockSpec((tm, K), lambda m: (m, 0)),
            pl.BlockSpec((K, N), lambda m: (0, 0)),
            pl.BlockSpec((1, N), lambda m: (0, 0)),
        ],
        out_specs=pl.BlockSpec((tm, N), lambda m: (m, 0)),
        compiler_params=pltpu.CompilerParams(
            dimension_semantics=("parallel",),
            vmem_limit_bytes=56 * 1024 * 1024),
        cost_estimate=pl.CostEstimate(
            flops=2 * M * K * N, transcendentals=M * N,
            bytes_accessed=M * K * 2 + K * N * 2 + M * N * 2),
    )(patches, w_stem, b_stem)


def _deconv1_kernel(aw_ref, b_ref, bias_ref, o_ref, *, H, W, Cin, Cout):
    py = pl.program_id(0)
    for px in range(2):
        acc = None
        for ai in range(2):
            a = py + ai
            asl = aw_ref[0, pl.ds(a, H), :, px * Cin:(px + 2) * Cin]
            A = asl.reshape(H * W, 2 * Cin)
            rs = pl.multiple_of((3 * a + px) * Cin, 8)
            d = jnp.dot(A, b_ref[pl.ds(rs, 2 * Cin),
                                 px * Cout:(px + 1) * Cout],
                        preferred_element_type=jnp.float32)
            acc = d if acc is None else acc + d
        r = jnp.maximum(acc + bias_ref[:, px * Cout:(px + 1) * Cout], 0.0)
        o_ref[0, :, px * Cout:(px + 1) * Cout] = r.astype(o_ref.dtype)


def _deconv1_layer(aw, B9, bias4, H, W):
    N = aw.shape[0]
    C3 = aw.shape[3]
    Cin = C3 // 3
    NC = B9.shape[1]
    Cout = NC // 4
    tn = NC // 2
    bias_row = bias4.reshape(1, NC)
    kern = functools.partial(_deconv1_kernel, H=H, W=W, Cin=Cin, Cout=Cout)
    flops = 2 * N * H * W * 8 * Cin * Cout
    bytes_accessed = (N * (H + 2) * W * C3 * 2 * 2 + 9 * Cin * NC * 2
                      + N * H * W * NC * 2)
    return pl.pallas_call(
        kern,
        out_shape=jax.ShapeDtypeStruct((N, H * W, NC), jnp.bfloat16),
        grid=(2, N),
        in_specs=[
            pl.BlockSpec((1, H + 2, W, C3), lambda n, i: (i, 0, 0, 0)),
            pl.BlockSpec((3 * C3, tn), lambda n, i: (0, n)),
            pl.BlockSpec((1, tn), lambda n, i: (0, n)),
        ],
        out_specs=pl.BlockSpec((1, H * W, tn), lambda n, i: (i, 0, n)),
        compiler_params=pltpu.CompilerParams(
            dimension_semantics=("parallel", "arbitrary"),
            vmem_limit_bytes=56 * 1024 * 1024),
        cost_estimate=pl.CostEstimate(
            flops=flops, transcendentals=0,
            bytes_accessed=int(bytes_accessed)),
    )(aw, B9, bias_row)


def _deconv2_kernel(aw_ref, b_ref, bias_ref, o_ref, *, H, W, Cin, Cout):
    C3 = 3 * Cin
    for ph in range(4):
        py, px = ph // 2, ph % 2
        acc = None
        for ai in range(2):
            a = py + ai
            asl = aw_ref[0, a:a + H, :, px * Cin:(px + 2) * Cin]
            A = asl.reshape(H * W, 2 * Cin)
            d = jnp.dot(A, b_ref[(3 * a + px) * Cin:(3 * a + px + 2) * Cin,
                                 ph * Cout:(ph + 1) * Cout],
                        preferred_element_type=jnp.float32)
            acc = d if acc is None else acc + d
        r = jnp.maximum(acc + bias_ref[:, ph * Cout:(ph + 1) * Cout], 0.0)
        o_ref[0, :, ph * Cout:(ph + 1) * Cout] = r.astype(o_ref.dtype)


def _deconv2_layer(aw, B9, bias4, H, W):
    N = aw.shape[0]
    C3 = aw.shape[3]
    Cin = C3 // 3
    NC = B9.shape[1]
    Cout = NC // 4
    bias_row = bias4.reshape(1, NC)
    kern = functools.partial(_deconv2_kernel, H=H, W=W, Cin=Cin, Cout=Cout)
    flops = 2 * N * H * W * 8 * Cin * Cout
    bytes_accessed = (N * (H + 2) * W * C3 * 2 + 9 * Cin * NC * 2
                      + N * H * W * NC * 2)
    return pl.pallas_call(
        kern,
        out_shape=jax.ShapeDtypeStruct((N, H * W, NC), jnp.bfloat16),
        grid=(N,),
        in_specs=[
            pl.BlockSpec((1, H + 2, W, C3), lambda i: (i, 0, 0, 0)),
            pl.BlockSpec((3 * C3, NC), lambda i: (0, 0)),
            pl.BlockSpec((1, NC), lambda i: (0, 0)),
        ],
        out_specs=pl.BlockSpec((1, H * W, NC), lambda i: (i, 0, 0)),
        compiler_params=pltpu.CompilerParams(
            dimension_semantics=("parallel",),
            vmem_limit_bytes=56 * 1024 * 1024),
        cost_estimate=pl.CostEstimate(
            flops=flops, transcendentals=0,
            bytes_accessed=int(bytes_accessed)),
    )(aw, B9, bias_row)


def _deconv_head_kernel(aw_ref, b_ref, bias_ref, w14_ref, o_ref, *, H, W):
    C3 = aw_ref.shape[3]
    Cout2 = b_ref.shape[1] // 2
    halves = []
    for py in range(2):
        acc = None
        for a in (py, py + 1):
            A = aw_ref[0, a:a + H].reshape(H * W, C3)
            d = jnp.dot(A, b_ref[a * C3:(a + 1) * C3,
                                 py * Cout2:(py + 1) * Cout2],
                        preferred_element_type=jnp.float32)
            acc = d if acc is None else acc + d
        halves.append(jnp.maximum(
            acc + bias_ref[:, py * Cout2:(py + 1) * Cout2], 0.0))
    r = jnp.concatenate(halves, axis=1).astype(jnp.bfloat16)
    g = jnp.dot(r, w14_ref[...], preferred_element_type=jnp.float32)
    o_ref[0] = g.astype(o_ref.dtype)


def _deconv_head_layer(aw, B9, bias4, w14, H, W):
    N = aw.shape[0]
    C3 = aw.shape[3]
    NC = B9.shape[1]
    GC = w14.shape[1]
    bias_row = bias4.reshape(1, NC)
    kern = functools.partial(_deconv_head_kernel, H=H, W=W)
    flops = 2 * N * H * W * (3 * C3 * NC + NC * GC)
    bytes_accessed = (N * (H + 2) * W * C3 * 2 + 3 * C3 * NC * 2
                      + N * H * W * GC * 2)
    return pl.pallas_call(
        kern,
        out_shape=jax.ShapeDtypeStruct((N, H * W, GC), jnp.bfloat16),
        grid=(N,),
        in_specs=[
            pl.BlockSpec((1, H + 2, W, C3), lambda i: (i, 0, 0, 0)),
            pl.BlockSpec((3 * C3, NC), lambda i: (0, 0)),
            pl.BlockSpec((1, NC), lambda i: (0, 0)),
            pl.BlockSpec((NC, GC), lambda i: (0, 0)),
        ],
        out_specs=pl.BlockSpec((1, H * W, GC), lambda i: (i, 0, 0)),
        compiler_params=pltpu.CompilerParams(
            dimension_semantics=("parallel",),
            vmem_limit_bytes=56 * 1024 * 1024),
        cost_estimate=pl.CostEstimate(
            flops=flops, transcendentals=0,
            bytes_accessed=int(bytes_accessed)),
    )(aw, B9, bias_row, w14)


def _resize_kernel(g_ref, whp0_ref, whp1_ref, wwp_ref, bias_ref, o_ref):
    c = pl.program_id(1)
    g = g_ref[0, 0]
    c0 = jnp.concatenate([g[0], g[1]], axis=-1)
    c1 = jnp.concatenate([g[2], g[3]], axis=-1)
    t = (jnp.dot(whp0_ref[...], c0, preferred_element_type=jnp.float32)
         + jnp.dot(whp1_ref[...], c1, preferred_element_type=jnp.float32))
    y = jnp.dot(t.astype(jnp.bfloat16), wwp_ref[...],
                preferred_element_type=jnp.float32)
    o_ref[0, 0] = y + bias_ref[c]


def _head_resize(ypp, wh, wwT, bias):
    N, C, _, h2, w2 = ypp.shape
    OH = wh.shape[0]
    OW = wwT.shape[1]
    whT = wh.reshape(OH, h2, 2).transpose(2, 0, 1)
    whp0 = whT[0]
    whp1 = whT[1]
    wwp = (wwT.reshape(w2, 2, OW).transpose(1, 0, 2)
           .reshape(2 * w2, OW))
    flops = 2 * N * C * (2 * OH * h2 * w2 * 2 + OH * 2 * w2 * OW)
    bytes_accessed = (N * C * 4 * h2 * w2 * 2 + 2 * OH * h2 * 2
                      + 2 * w2 * OW * 2 + N * C * OH * OW * 4)
    return pl.pallas_call(
        _resize_kernel,
        out_shape=jax.ShapeDtypeStruct((N, C, OH, OW), jnp.float32),
        grid=(N, C),
        in_specs=[
            pl.BlockSpec((1, 1, 4, h2, w2), lambda n, c: (n, c, 0, 0, 0)),
            pl.BlockSpec((OH, h2), lambda n, c: (0, 0)),
            pl.BlockSpec((OH, h2), lambda n, c: (0, 0)),
            pl.BlockSpec((2 * w2, OW), lambda n, c: (0, 0)),
            pl.BlockSpec(memory_space=pltpu.MemorySpace.SMEM),
        ],
        out_specs=pl.BlockSpec((1, 1, OH, OW), lambda n, c: (n, c, 0, 0)),
        compiler_params=pltpu.CompilerParams(
            dimension_semantics=("parallel", "parallel"),
            vmem_limit_bytes=56 * 1024 * 1024),
        cost_estimate=pl.CostEstimate(
            flops=flops, transcendentals=0,
            bytes_accessed=int(bytes_accessed)),
    )(ypp, whp0, whp1, wwp, bias.astype(jnp.float32))


def _width_cat(x_nhwc):
    xp = jnp.pad(x_nhwc, ((0, 0), (1, 1), (1, 1), (0, 0)))
    W = x_nhwc.shape[2]
    return jnp.concatenate([xp[:, :, b:b + W, :] for b in range(3)], axis=-1)


def _depth_to_space(y, H, W, C):
    N = y.shape[0]
    return (y.reshape(N, H, W, 2, 2, C)
             .transpose(0, 1, 3, 2, 4, 5)
             .reshape(N, 2 * H, 2 * W, C))


def kernel(x, w_stem, b_stem, w1x1, b1x1, Wh, WwT,
           B_t1, bias4_t1, B_t2, bias4_t2, B_t3, bias4_t3):
    N, C, H, W = x.shape
    P = 32
    hp, wp = H // P, W // P

    xb = x.astype(jnp.bfloat16).reshape(N, C, hp, P, wp, P)
    x1 = jax.lax.optimization_barrier(xb.transpose(0, 2, 1, 3, 4, 5))
    patches = (x1.transpose(0, 1, 4, 2, 3, 5)
               .reshape(N * hp * wp, C * P * P))
    f0 = _stem_matmul(patches, w_stem, b_stem).reshape(N, hp, wp, -1)

    aw1 = _width_cat(f0)
    y1 = _deconv1_layer(aw1, B_t1, bias4_t1, hp, wp)
    f1 = _depth_to_space(y1, hp, wp, B_t1.shape[1] // 4)

    aw2 = _width_cat(f1)
    y2 = _deconv2_layer(aw2, B_t2, bias4_t2, 2 * hp, 2 * wp)
    f2 = _depth_to_space(y2, 2 * hp, 2 * wp, B_t2.shape[1] // 4)

    aw3 = _width_cat(f2)
    nclass = w1x1.shape[1]
    cout3 = B_t3.shape[1] // 4
    eye4 = jnp.eye(4, dtype=jnp.bfloat16)
    w14 = (eye4[:, None, None, :] * w1x1[None, :, :, None]
           ).reshape(4 * cout3, 4 * nclass)
    y3 = _deconv_head_layer(aw3, B_t3, bias4_t3, w14, 4 * hp, 4 * wp)

    ypp = (y3.reshape(N, 4 * hp, 4 * wp, nclass, 4)
             .transpose(0, 3, 4, 1, 2))
    return _head_resize(ypp, Wh, WwT, b1x1)


# speedup vs baseline: 1.9788x; 1.2033x over previous
import functools

import jax
import jax.numpy as jnp
from jax.experimental import pallas as pl
from jax.experimental.pallas import tpu as pltpu


def _stem_kernel(a_ref, b_ref, bias_ref, o_ref):
    r = jnp.dot(a_ref[...], b_ref[...], preferred_element_type=jnp.float32)
    r = r + bias_ref[...]
    r = r * jax.nn.sigmoid(r)
    o_ref[...] = r.astype(o_ref.dtype)


def _stem_matmul(patches, w_stem, b_stem):
    M, K = patches.shape
    N = w_stem.shape[1]
    b_stem = b_stem.reshape(1, N)
    tm = min(512, M)
    grid = (M // tm,)
    return pl.pallas_call(
        _stem_kernel,
        out_shape=jax.ShapeDtypeStruct((M, N), jnp.bfloat16),
        grid=grid,
        in_specs=[
            pl.BlockSpec((tm, K), lambda m: (m, 0)),
            pl.BlockSpec((K, N), lambda m: (0, 0)),
            pl.BlockSpec((1, N), lambda m: (0, 0)),
        ],
        out_specs=pl.BlockSpec((tm, N), lambda m: (m, 0)),
        compiler_params=pltpu.CompilerParams(
            dimension_semantics=("parallel",),
            vmem_limit_bytes=56 * 1024 * 1024),
        cost_estimate=pl.CostEstimate(
            flops=2 * M * K * N, transcendentals=M * N,
            bytes_accessed=M * K * 2 + K * N * 2 + M * N * 2),
    )(patches, w_stem, b_stem)


def _deconv1_kernel(xp_ref, b_ref, bias_ref, o_ref, *, H, W, Cin, Cout):
    py = pl.program_id(0)
    for nb in range(xp_ref.shape[0]):
        for px in range(2):
            acc = None
            for ai in range(2):
                a = py + ai
                for b in (px, px + 1):
                    A = xp_ref[nb, pl.ds(a, H), b:b + W, :].reshape(H * W, Cin)
                    rs = pl.multiple_of((3 * a + b) * Cin, 8)
                    d = jnp.dot(A, b_ref[pl.ds(rs, Cin),
                                         px * Cout:(px + 1) * Cout],
                                preferred_element_type=jnp.float32)
                    acc = d if acc is None else acc + d
            r = jnp.maximum(acc + bias_ref[:, px * Cout:(px + 1) * Cout], 0.0)
            o_ref[nb, :, px * Cout:(px + 1) * Cout] = r.astype(o_ref.dtype)


def _deconv1_layer(xp, B9, bias4, H, W):
    N = xp.shape[0]
    Cin = xp.shape[3]
    NC = B9.shape[1]
    Cout = NC // 4
    tn = NC // 2
    bias_row = bias4.reshape(1, NC)
    kern = functools.partial(_deconv1_kernel, H=H, W=W, Cin=Cin, Cout=Cout)
    flops = 2 * N * H * W * 8 * Cin * Cout
    bytes_accessed = (N * (H + 2) * (W + 2) * Cin * 2 * 2 + 9 * Cin * NC * 2
                      + N * H * W * NC * 2)
    NB = 2 if N % 2 == 0 else 1
    return pl.pallas_call(
        kern,
        out_shape=jax.ShapeDtypeStruct((N, H * W, NC), jnp.bfloat16),
        grid=(2, N // NB),
        in_specs=[
            pl.BlockSpec((NB, H + 2, W + 2, Cin), lambda n, i: (i, 0, 0, 0)),
            pl.BlockSpec((9 * Cin, tn), lambda n, i: (0, n)),
            pl.BlockSpec((1, tn), lambda n, i: (0, n)),
        ],
        out_specs=pl.BlockSpec((NB, H * W, tn), lambda n, i: (i, 0, n)),
        compiler_params=pltpu.CompilerParams(
            dimension_semantics=("parallel", "arbitrary"),
            vmem_limit_bytes=56 * 1024 * 1024),
        cost_estimate=pl.CostEstimate(
            flops=flops, transcendentals=0,
            bytes_accessed=int(bytes_accessed)),
    )(xp, B9, bias_row)


def _deconv2_kernel(xp_ref, b_ref, bias_ref, o_ref, *, H, W, Cin, Cout):
    for nb in range(xp_ref.shape[0]):
        for ph in range(4):
            py, px = ph // 2, ph % 2
            acc = None
            for a in (py, py + 1):
                for b in (px, px + 1):
                    A = xp_ref[nb, a:a + H, b:b + W, :].reshape(H * W, Cin)
                    d = jnp.dot(
                        A, b_ref[(3 * a + b) * Cin:(3 * a + b + 1) * Cin,
                                 ph * Cout:(ph + 1) * Cout],
                        preferred_element_type=jnp.float32)
                    acc = d if acc is None else acc + d
            r = jnp.maximum(acc + bias_ref[:, ph * Cout:(ph + 1) * Cout], 0.0)
            o_ref[nb, :, ph * Cout:(ph + 1) * Cout] = r.astype(o_ref.dtype)


def _deconv2_layer(xp, B9, bias4, H, W):
    N = xp.shape[0]
    Cin = xp.shape[3]
    NC = B9.shape[1]
    Cout = NC // 4
    bias_row = bias4.reshape(1, NC)
    kern = functools.partial(_deconv2_kernel, H=H, W=W, Cin=Cin, Cout=Cout)
    flops = 2 * N * H * W * 8 * Cin * Cout
    bytes_accessed = (N * (H + 2) * (W + 2) * Cin * 2 + 9 * Cin * NC * 2
                      + N * H * W * NC * 2)
    NB = 2 if N % 2 == 0 else 1
    return pl.pallas_call(
        kern,
        out_shape=jax.ShapeDtypeStruct((N, H * W, NC), jnp.bfloat16),
        grid=(N // NB,),
        in_specs=[
            pl.BlockSpec((NB, H + 2, W + 2, Cin), lambda i: (i, 0, 0, 0)),
            pl.BlockSpec((9 * Cin, NC), lambda i: (0, 0)),
            pl.BlockSpec((1, NC), lambda i: (0, 0)),
        ],
        out_specs=pl.BlockSpec((NB, H * W, NC), lambda i: (i, 0, 0)),
        compiler_params=pltpu.CompilerParams(
            dimension_semantics=("parallel",),
            vmem_limit_bytes=56 * 1024 * 1024),
        cost_estimate=pl.CostEstimate(
            flops=flops, transcendentals=0,
            bytes_accessed=int(bytes_accessed)),
    )(xp, B9, bias_row)


def _deconv_head_kernel(xp_ref, b_ref, bias_ref, w14_ref, o_ref, *, H, W,
                        Cin):
    Cout2 = b_ref.shape[1] // 2
    for nb in range(xp_ref.shape[0]):
        halves = []
        for py in range(2):
            acc = None
            for a in (py, py + 1):
                for b in range(3):
                    A = xp_ref[nb, a:a + H, b:b + W, :].reshape(H * W, Cin)
                    d = jnp.dot(
                        A, b_ref[(3 * a + b) * Cin:(3 * a + b + 1) * Cin,
                                 py * Cout2:(py + 1) * Cout2],
                        preferred_element_type=jnp.float32)
                    acc = d if acc is None else acc + d
            halves.append(jnp.maximum(
                acc + bias_ref[:, py * Cout2:(py + 1) * Cout2], 0.0))
        r = jnp.concatenate(halves, axis=1).astype(jnp.bfloat16)
        g = jnp.dot(r, w14_ref[...], preferred_element_type=jnp.float32)
        o_ref[nb] = g.astype(o_ref.dtype)


def _deconv_head_layer(xp, B9, bias4, w14, H, W):
    N = xp.shape[0]
    Cin = xp.shape[3]
    NC = B9.shape[1]
    GC = w14.shape[1]
    bias_row = bias4.reshape(1, NC)
    kern = functools.partial(_deconv_head_kernel, H=H, W=W, Cin=Cin)
    flops = 2 * N * H * W * (6 * Cin * NC // 2 * 2 + NC * GC)
    bytes_accessed = (N * (H + 2) * (W + 2) * Cin * 2 + 9 * Cin * NC * 2
                      + N * H * W * GC * 2)
    NB = 2 if N % 2 == 0 else 1
    return pl.pallas_call(
        kern,
        out_shape=jax.ShapeDtypeStruct((N, H * W, GC), jnp.bfloat16),
        grid=(N // NB,),
        in_specs=[
            pl.BlockSpec((NB, H + 2, W + 2, Cin), lambda i: (i, 0, 0, 0)),
            pl.BlockSpec((9 * Cin, NC), lambda i: (0, 0)),
            pl.BlockSpec((1, NC), lambda i: (0, 0)),
            pl.BlockSpec((NC, GC), lambda i: (0, 0)),
        ],
        out_specs=pl.BlockSpec((NB, H * W, GC), lambda i: (i, 0, 0)),
        compiler_params=pltpu.CompilerParams(
            dimension_semantics=("parallel",),
            vmem_limit_bytes=56 * 1024 * 1024),
        cost_estimate=pl.CostEstimate(
            flops=flops, transcendentals=0,
            bytes_accessed=int(bytes_accessed)),
    )(xp, B9, bias_row, w14)


def _resize_kernel(g_ref, whp0_ref, whp1_ref, wwp_ref, bias_ref, o_ref):
    c = pl.program_id(1)
    for nb in range(g_ref.shape[0]):
        g = g_ref[nb, 0]
        c0 = jnp.concatenate([g[0], g[1]], axis=-1)
        c1 = jnp.concatenate([g[2], g[3]], axis=-1)
        t = (jnp.dot(whp0_ref[...], c0, preferred_element_type=jnp.float32)
             + jnp.dot(whp1_ref[...], c1, preferred_element_type=jnp.float32))
        y = jnp.dot(t.astype(jnp.bfloat16), wwp_ref[...],
                    preferred_element_type=jnp.float32)
        o_ref[nb, 0] = y + bias_ref[c]


def _head_resize(ypp, wh, wwT, bias):
    N, C, _, h2, w2 = ypp.shape
    OH = wh.shape[0]
    OW = wwT.shape[1]
    whT = wh.reshape(OH, h2, 2).transpose(2, 0, 1)
    whp0 = whT[0]
    whp1 = whT[1]
    wwp = (wwT.reshape(w2, 2, OW).transpose(1, 0, 2)
           .reshape(2 * w2, OW))
    flops = 2 * N * C * (2 * OH * h2 * w2 * 2 + OH * 2 * w2 * OW)
    bytes_accessed = (N * C * 4 * h2 * w2 * 2 + 2 * OH * h2 * 2
                      + 2 * w2 * OW * 2 + N * C * OH * OW * 4)
    NB = 4 if N % 4 == 0 else 1
    return pl.pallas_call(
        _resize_kernel,
        out_shape=jax.ShapeDtypeStruct((N, C, OH, OW), jnp.float32),
        grid=(N // NB, C),
        in_specs=[
            pl.BlockSpec((NB, 1, 4, h2, w2), lambda n, c: (n, c, 0, 0, 0)),
            pl.BlockSpec((OH, h2), lambda n, c: (0, 0)),
            pl.BlockSpec((OH, h2), lambda n, c: (0, 0)),
            pl.BlockSpec((2 * w2, OW), lambda n, c: (0, 0)),
            pl.BlockSpec(memory_space=pltpu.MemorySpace.SMEM),
        ],
        out_specs=pl.BlockSpec((NB, 1, OH, OW), lambda n, c: (n, c, 0, 0)),
        compiler_params=pltpu.CompilerParams(
            dimension_semantics=("parallel", "parallel"),
            vmem_limit_bytes=56 * 1024 * 1024),
        cost_estimate=pl.CostEstimate(
            flops=flops, transcendentals=0,
            bytes_accessed=int(bytes_accessed)),
    )(ypp, whp0, whp1, wwp, bias.astype(jnp.float32))


def _ring_pad(x_nhwc):
    return jnp.pad(x_nhwc, ((0, 0), (1, 1), (1, 1), (0, 0)))


def _depth_to_space(y, H, W, C):
    N = y.shape[0]
    return (y.reshape(N, H, W, 2, 2, C)
             .transpose(0, 1, 3, 2, 4, 5)
             .reshape(N, 2 * H, 2 * W, C))


def kernel(x, w_stem, b_stem, w1x1, b1x1, Wh, WwT,
           B_t1, bias4_t1, B_t2, bias4_t2, B_t3, bias4_t3):
    N, C, H, W = x.shape
    P = 32
    hp, wp = H // P, W // P

    xb = x.astype(jnp.bfloat16).reshape(N, C, hp, P, wp, P)
    x1 = jax.lax.optimization_barrier(xb.transpose(0, 2, 1, 3, 4, 5))
    patches = (x1.transpose(0, 1, 4, 2, 3, 5)
               .reshape(N * hp * wp, C * P * P))
    f0 = _stem_matmul(patches, w_stem, b_stem).reshape(N, hp, wp, -1)

    xp1 = _ring_pad(f0)
    y1 = _deconv1_layer(xp1, B_t1, bias4_t1, hp, wp)
    f1 = _depth_to_space(y1, hp, wp, B_t1.shape[1] // 4)

    xp2 = _ring_pad(f1)
    y2 = _deconv2_layer(xp2, B_t2, bias4_t2, 2 * hp, 2 * wp)
    f2 = _depth_to_space(y2, 2 * hp, 2 * wp, B_t2.shape[1] // 4)

    xp3 = _ring_pad(f2)
    nclass = w1x1.shape[1]
    cout3 = B_t3.shape[1] // 4
    eye4 = jnp.eye(4, dtype=jnp.bfloat16)
    w14 = (eye4[:, None, None, :] * w1x1[None, :, :, None]
           ).reshape(4 * cout3, 4 * nclass)
    y3 = _deconv_head_layer(xp3, B_t3, bias4_t3, w14, 4 * hp, 4 * wp)

    ypp = (y3.reshape(N, 4 * hp, 4 * wp, nclass, 4)
             .transpose(0, 3, 4, 1, 2))
    return _head_resize(ypp, Wh, WwT, b1x1)


# speedup vs baseline: 1.9806x; 1.0009x over previous
import functools

import jax
import jax.numpy as jnp
from jax.experimental import pallas as pl
from jax.experimental.pallas import tpu as pltpu


def _stem_kernel(a_ref, b_ref, bias_ref, o_ref):
    r = jnp.dot(a_ref[...], b_ref[...], preferred_element_type=jnp.float32)
    r = r + bias_ref[...]
    r = r * jax.nn.sigmoid(r)
    o_ref[...] = r.astype(o_ref.dtype)


def _stem_matmul(patches, w_stem, b_stem):
    M, K = patches.shape
    N = w_stem.shape[1]
    b_stem = b_stem.reshape(1, N)
    tm = min(256, M)
    grid = (M // tm,)
    return pl.pallas_call(
        _stem_kernel,
        out_shape=jax.ShapeDtypeStruct((M, N), jnp.bfloat16),
        grid=grid,
        in_specs=[
            pl.BlockSpec((tm, K), lambda m: (m, 0)),
            pl.BlockSpec((K, N), lambda m: (0, 0)),
            pl.BlockSpec((1, N), lambda m: (0, 0)),
        ],
        out_specs=pl.BlockSpec((tm, N), lambda m: (m, 0)),
        compiler_params=pltpu.CompilerParams(
            dimension_semantics=("parallel",),
            vmem_limit_bytes=56 * 1024 * 1024),
        cost_estimate=pl.CostEstimate(
            flops=2 * M * K * N, transcendentals=M * N,
            bytes_accessed=M * K * 2 + K * N * 2 + M * N * 2),
    )(patches, w_stem, b_stem)


def _deconv1_kernel(xp_ref, b_ref, bias_ref, o_ref, *, H, W, Cin, Cout):
    py = pl.program_id(0)
    for nb in range(xp_ref.shape[0]):
        for px in range(2):
            acc = None
            for ai in range(2):
                a = py + ai
                for b in (px, px + 1):
                    A = xp_ref[nb, pl.ds(a, H), b:b + W, :].reshape(H * W, Cin)
                    rs = pl.multiple_of((3 * a + b) * Cin, 8)
                    d = jnp.dot(A, b_ref[pl.ds(rs, Cin),
                                         px * Cout:(px + 1) * Cout],
                                preferred_element_type=jnp.float32)
                    acc = d if acc is None else acc + d
            r = jnp.maximum(acc + bias_ref[:, px * Cout:(px + 1) * Cout], 0.0)
            o_ref[nb, :, px * Cout:(px + 1) * Cout] = r.astype(o_ref.dtype)


def _deconv1_layer(xp, B9, bias4, H, W):
    N = xp.shape[0]
    Cin = xp.shape[3]
    NC = B9.shape[1]
    Cout = NC // 4
    tn = NC // 2
    bias_row = bias4.reshape(1, NC)
    kern = functools.partial(_deconv1_kernel, H=H, W=W, Cin=Cin, Cout=Cout)
    flops = 2 * N * H * W * 8 * Cin * Cout
    bytes_accessed = (N * (H + 2) * (W + 2) * Cin * 2 * 2 + 9 * Cin * NC * 2
                      + N * H * W * NC * 2)
    NB = 2 if N % 2 == 0 else 1
    return pl.pallas_call(
        kern,
        out_shape=jax.ShapeDtypeStruct((N, H * W, NC), jnp.bfloat16),
        grid=(2, N // NB),
        in_specs=[
            pl.BlockSpec((NB, H + 2, W + 2, Cin), lambda n, i: (i, 0, 0, 0)),
            pl.BlockSpec((9 * Cin, tn), lambda n, i: (0, n)),
            pl.BlockSpec((1, tn), lambda n, i: (0, n)),
        ],
        out_specs=pl.BlockSpec((NB, H * W, tn), lambda n, i: (i, 0, n)),
        compiler_params=pltpu.CompilerParams(
            dimension_semantics=("parallel", "arbitrary"),
            vmem_limit_bytes=56 * 1024 * 1024),
        cost_estimate=pl.CostEstimate(
            flops=flops, transcendentals=0,
            bytes_accessed=int(bytes_accessed)),
    )(xp, B9, bias_row)


def _deconv2_kernel(xp_ref, b_ref, bias_ref, o_ref, *, H, W, Cin, Cout):
    for nb in range(xp_ref.shape[0]):
        for ph in range(4):
            py, px = ph // 2, ph % 2
            acc = None
            for a in (py, py + 1):
                for b in (px, px + 1):
                    A = xp_ref[nb, a:a + H, b:b + W, :].reshape(H * W, Cin)
                    d = jnp.dot(
                        A, b_ref[(3 * a + b) * Cin:(3 * a + b + 1) * Cin,
                                 ph * Cout:(ph + 1) * Cout],
                        preferred_element_type=jnp.float32)
                    acc = d if acc is None else acc + d
            r = jnp.maximum(acc + bias_ref[:, ph * Cout:(ph + 1) * Cout], 0.0)
            o_ref[nb, :, ph * Cout:(ph + 1) * Cout] = r.astype(o_ref.dtype)


def _deconv2_layer(xp, B9, bias4, H, W):
    N = xp.shape[0]
    Cin = xp.shape[3]
    NC = B9.shape[1]
    Cout = NC // 4
    bias_row = bias4.reshape(1, NC)
    kern = functools.partial(_deconv2_kernel, H=H, W=W, Cin=Cin, Cout=Cout)
    flops = 2 * N * H * W * 8 * Cin * Cout
    bytes_accessed = (N * (H + 2) * (W + 2) * Cin * 2 + 9 * Cin * NC * 2
                      + N * H * W * NC * 2)
    NB = 2 if N % 2 == 0 else 1
    return pl.pallas_call(
        kern,
        out_shape=jax.ShapeDtypeStruct((N, H * W, NC), jnp.bfloat16),
        grid=(N // NB,),
        in_specs=[
            pl.BlockSpec((NB, H + 2, W + 2, Cin), lambda i: (i, 0, 0, 0)),
            pl.BlockSpec((9 * Cin, NC), lambda i: (0, 0)),
            pl.BlockSpec((1, NC), lambda i: (0, 0)),
        ],
        out_specs=pl.BlockSpec((NB, H * W, NC), lambda i: (i, 0, 0)),
        compiler_params=pltpu.CompilerParams(
            dimension_semantics=("parallel",),
            vmem_limit_bytes=56 * 1024 * 1024),
        cost_estimate=pl.CostEstimate(
            flops=flops, transcendentals=0,
            bytes_accessed=int(bytes_accessed)),
    )(xp, B9, bias_row)


def _deconv_head_kernel(xp_ref, b_ref, bias_ref, w14_ref, o_ref, *, H, W,
                        Cin):
    Cout2 = b_ref.shape[1] // 2
    for nb in range(xp_ref.shape[0]):
        halves = []
        for py in range(2):
            acc = None
            for a in (py, py + 1):
                for b in range(3):
                    A = xp_ref[nb, a:a + H, b:b + W, :].reshape(H * W, Cin)
                    d = jnp.dot(
                        A, b_ref[(3 * a + b) * Cin:(3 * a + b + 1) * Cin,
                                 py * Cout2:(py + 1) * Cout2],
                        preferred_element_type=jnp.float32)
                    acc = d if acc is None else acc + d
            halves.append(jnp.maximum(
                acc + bias_ref[:, py * Cout2:(py + 1) * Cout2], 0.0))
        r = jnp.concatenate(halves, axis=1).astype(jnp.bfloat16)
        g = jnp.dot(r, w14_ref[...], preferred_element_type=jnp.float32)
        o_ref[nb] = g.astype(o_ref.dtype)


def _deconv_head_layer(xp, B9, bias4, w14, H, W):
    N = xp.shape[0]
    Cin = xp.shape[3]
    NC = B9.shape[1]
    GC = w14.shape[1]
    bias_row = bias4.reshape(1, NC)
    kern = functools.partial(_deconv_head_kernel, H=H, W=W, Cin=Cin)
    flops = 2 * N * H * W * (6 * Cin * NC // 2 * 2 + NC * GC)
    bytes_accessed = (N * (H + 2) * (W + 2) * Cin * 2 + 9 * Cin * NC * 2
                      + N * H * W * GC * 2)
    NB = 2 if N % 2 == 0 else 1
    return pl.pallas_call(
        kern,
        out_shape=jax.ShapeDtypeStruct((N, H * W, GC), jnp.bfloat16),
        grid=(N // NB,),
        in_specs=[
            pl.BlockSpec((NB, H + 2, W + 2, Cin), lambda i: (i, 0, 0, 0)),
            pl.BlockSpec((9 * Cin, NC), lambda i: (0, 0)),
            pl.BlockSpec((1, NC), lambda i: (0, 0)),
            pl.BlockSpec((NC, GC), lambda i: (0, 0)),
        ],
        out_specs=pl.BlockSpec((NB, H * W, GC), lambda i: (i, 0, 0)),
        compiler_params=pltpu.CompilerParams(
            dimension_semantics=("parallel",),
            vmem_limit_bytes=56 * 1024 * 1024),
        cost_estimate=pl.CostEstimate(
            flops=flops, transcendentals=0,
            bytes_accessed=int(bytes_accessed)),
    )(xp, B9, bias_row, w14)


def _resize_kernel(g_ref, whp0_ref, whp1_ref, wwp_ref, bias_ref, o_ref):
    c = pl.program_id(1)
    for nb in range(g_ref.shape[0]):
        g = g_ref[nb, 0]
        c0 = jnp.concatenate([g[0], g[1]], axis=-1)
        c1 = jnp.concatenate([g[2], g[3]], axis=-1)
        t = (jnp.dot(whp0_ref[...], c0, preferred_element_type=jnp.float32)
             + jnp.dot(whp1_ref[...], c1, preferred_element_type=jnp.float32))
        y = jnp.dot(t.astype(jnp.bfloat16), wwp_ref[...],
                    preferred_element_type=jnp.float32)
        o_ref[nb, 0] = y + bias_ref[c]


def _head_resize(ypp, wh, wwT, bias):
    N, C, _, h2, w2 = ypp.shape
    OH = wh.shape[0]
    OW = wwT.shape[1]
    whT = wh.reshape(OH, h2, 2).transpose(2, 0, 1)
    whp0 = whT[0]
    whp1 = whT[1]
    wwp = (wwT.reshape(w2, 2, OW).transpose(1, 0, 2)
           .reshape(2 * w2, OW))
    flops = 2 * N * C * (2 * OH * h2 * w2 * 2 + OH * 2 * w2 * OW)
    bytes_accessed = (N * C * 4 * h2 * w2 * 2 + 2 * OH * h2 * 2
                      + 2 * w2 * OW * 2 + N * C * OH * OW * 4)
    NB = 4 if N % 4 == 0 else 1
    return pl.pallas_call(
        _resize_kernel,
        out_shape=jax.ShapeDtypeStruct((N, C, OH, OW), jnp.float32),
        grid=(N // NB, C),
        in_specs=[
            pl.BlockSpec((NB, 1, 4, h2, w2), lambda n, c: (n, c, 0, 0, 0)),
            pl.BlockSpec((OH, h2), lambda n, c: (0, 0)),
            pl.BlockSpec((OH, h2), lambda n, c: (0, 0)),
            pl.BlockSpec((2 * w2, OW), lambda n, c: (0, 0)),
            pl.BlockSpec(memory_space=pltpu.MemorySpace.SMEM),
        ],
        out_specs=pl.BlockSpec((NB, 1, OH, OW), lambda n, c: (n, c, 0, 0)),
        compiler_params=pltpu.CompilerParams(
            dimension_semantics=("parallel", "parallel"),
            vmem_limit_bytes=56 * 1024 * 1024),
        cost_estimate=pl.CostEstimate(
            flops=flops, transcendentals=0,
            bytes_accessed=int(bytes_accessed)),
    )(ypp, whp0, whp1, wwp, bias.astype(jnp.float32))


def _ring_pad(x_nhwc):
    return jnp.pad(x_nhwc, ((0, 0), (1, 1), (1, 1), (0, 0)))


def _depth_to_space(y, H, W, C):
    N = y.shape[0]
    return (y.reshape(N, H, W, 2, 2, C)
             .transpose(0, 1, 3, 2, 4, 5)
             .reshape(N, 2 * H, 2 * W, C))


def kernel(x, w_stem, b_stem, w1x1, b1x1, Wh, WwT,
           B_t1, bias4_t1, B_t2, bias4_t2, B_t3, bias4_t3):
    N, C, H, W = x.shape
    P = 32
    hp, wp = H // P, W // P

    xb = x.astype(jnp.bfloat16).reshape(N, C, hp, P, wp, P)
    x1 = jax.lax.optimization_barrier(xb.transpose(0, 2, 1, 3, 4, 5))
    patches = (x1.transpose(0, 1, 4, 2, 3, 5)
               .reshape(N * hp * wp, C * P * P))
    f0 = _stem_matmul(patches, w_stem, b_stem).reshape(N, hp, wp, -1)

    xp1 = _ring_pad(f0)
    y1 = _deconv1_layer(xp1, B_t1, bias4_t1, hp, wp)
    f1 = _depth_to_space(y1, hp, wp, B_t1.shape[1] // 4)

    xp2 = _ring_pad(f1)
    y2 = _deconv2_layer(xp2, B_t2, bias4_t2, 2 * hp, 2 * wp)
    f2 = _depth_to_space(y2, 2 * hp, 2 * wp, B_t2.shape[1] // 4)

    xp3 = _ring_pad(f2)
    nclass = w1x1.shape[1]
    cout3 = B_t3.shape[1] // 4
    eye4 = jnp.eye(4, dtype=jnp.bfloat16)
    w14 = (eye4[:, None, None, :] * w1x1[None, :, :, None]
           ).reshape(4 * cout3, 4 * nclass)
    y3 = _deconv_head_layer(xp3, B_t3, bias4_t3, w14, 4 * hp, 4 * wp)

    ypp = (y3.reshape(N, 4 * hp, 4 * wp, nclass, 4)
             .transpose(0, 3, 4, 1, 2))
    return _head_resize(ypp, Wh, WwT, b1x1)
